# baseline (speedup 1.0000x reference)
import functools

import jax
import jax.numpy as jnp
from jax import lax
from jax.experimental import pallas as pl
from jax.experimental.pallas import tpu as pltpu, tpu_sc as plsc

VOCAB = 1000000
EMBED_DIM = 32
BATCH = 16384

_info = plsc.get_sparse_core_info()
_NC, _NS = _info.num_cores, _info.num_subcores
_NW = _NC * _NS
_BPW = BATCH // _NW
_CHUNK = 128
_NCHUNK = _BPW // _CHUNK
_BLK = 8


def _make_gather():
    mesh = plsc.VectorSubcoreMesh(core_axis_name="c", subcore_axis_name="s")

    @functools.partial(
        pl.kernel,
        mesh=mesh,
        compiler_params=pltpu.CompilerParams(use_tc_tiling_on_sc=False),
        out_type=jax.ShapeDtypeStruct(
            (_NW, _NCHUNK, _CHUNK, _BLK, EMBED_DIM), jnp.float32
        ),
        scratch_types=[
            pltpu.VMEM((_NCHUNK, _CHUNK), jnp.int32),
            pltpu.VMEM((2, _CHUNK, _BLK, EMBED_DIM), jnp.float32),
            pltpu.SemaphoreType.DMA,
            pltpu.SemaphoreType.DMA,
        ],
    )
    def gather_kernel(idx_hbm, table_hbm, out_hbm, idx_v, rows_v, sem0, sem1):
        wid = lax.axis_index("s") * _NC + lax.axis_index("c")
        pltpu.sync_copy(idx_hbm.at[wid], idx_v)
        sems = [sem0, sem1]
        cps = [
            pltpu.async_copy(table_hbm.at[idx_v.at[c]], rows_v.at[c], sems[c])
            for c in range(2)
        ]
        for c in range(_NCHUNK):
            b = c % 2
            cps[b].wait()
            pltpu.sync_copy(rows_v.at[b], out_hbm.at[wid, c])
            nxt = c + 2
            if nxt < _NCHUNK:
                cps[b] = pltpu.async_copy(
                    table_hbm.at[idx_v.at[nxt]], rows_v.at[b], sems[b]
                )

    return gather_kernel


_gather = _make_gather()


def kernel(anime_id, embedding_table):
    idx = anime_id.astype(jnp.int32)
    blk = (idx >> 3).reshape(_NW, _NCHUNK, _CHUNK)
    sub = (idx & 7).reshape(BATCH, 1, 1)
    table3 = embedding_table.reshape(VOCAB // _BLK, _BLK, EMBED_DIM)
    blocks = _gather(blk, table3).reshape(BATCH, _BLK, EMBED_DIM)
    return jnp.take_along_axis(blocks, sub, axis=1).reshape(BATCH, EMBED_DIM)
